# Initial kernel scaffold; baseline (speedup 1.0000x reference)
import jax
import jax.numpy as jnp
from jax.experimental import pallas as pl


def kernel(logits, uniform_noise):
    raise NotImplementedError("write your pallas kernel here")


# speedup vs baseline: 2.2260x; 2.2260x over previous
import jax
import jax.numpy as jnp
from jax.experimental import pallas as pl

_N = 8192
_ROWS_PER_BLOCK = 256


def _gumbel_softmax_block(l_ref, u_ref, o_ref):
    g = -jnp.log(-jnp.log(u_ref[...]))
    z = l_ref[...] + g
    m = jnp.max(z, axis=-1, keepdims=True)
    e = jnp.exp(z - m)
    s = jnp.sum(e, axis=-1, keepdims=True)
    o_ref[...] = e * (1.0 / s)


def kernel(logits, uniform_noise):
    n = logits.shape[0]
    rows = _ROWS_PER_BLOCK
    grid = (n // rows,)
    spec = pl.BlockSpec((rows, logits.shape[1]), lambda i: (i, 0))
    return pl.pallas_call(
        _gumbel_softmax_block,
        grid=grid,
        in_specs=[spec, spec],
        out_specs=spec,
        out_shape=jax.ShapeDtypeStruct(logits.shape, logits.dtype),
    )(logits, uniform_noise)
